# Initial kernel scaffold; baseline (speedup 1.0000x reference)
import jax
import jax.numpy as jnp
from jax.experimental import pallas as pl


def kernel(x, centers, transform_inds):
    raise NotImplementedError("write your pallas kernel here")


# speedup vs baseline: 3.4261x; 3.4261x over previous
import functools

import jax
import jax.numpy as jnp
from jax import lax
from jax.experimental import pallas as pl
from jax.experimental.pallas import tpu as pltpu
from jax.experimental.pallas import tpu_sc as plsc

NC = 2
NS = 16
L = 16
NW = NC * NS

CPAD = 1024
CHUNK = CPAD // NW

_mesh = plsc.VectorSubcoreMesh(core_axis_name="c", subcore_axis_name="s")
_params = pltpu.CompilerParams(needs_layout_passes=False)


@functools.partial(
    pl.kernel,
    out_type=jax.ShapeDtypeStruct((CPAD,), jnp.float32),
    mesh=_mesh,
    compiler_params=_params,
    scratch_types=[
        pltpu.VMEM((CPAD,), jnp.float32),
        pltpu.VMEM((CHUNK,), jnp.int32),
    ],
)
def _sort_centers(c_hbm, out_hbm, c_v, rank_v):
    wid = lax.axis_index("s") * NC + lax.axis_index("c")
    pltpu.sync_copy(c_hbm, c_v)
    base = wid * CHUNK
    iota = lax.iota(jnp.int32, L)
    j0 = base + iota
    j1 = base + L + iota
    v0 = c_v[pl.ds(base, L)]
    v1 = c_v[pl.ds(base + L, L)]

    def _splat(k):
        return plsc.load_gather(c_v, [jnp.full((L,), k, jnp.int32)])

    def count_le(k, r):
        ck = _splat(k)
        return (r[0] + (ck <= v0).astype(jnp.int32),
                r[1] + (ck <= v1).astype(jnp.int32))

    def count_lt(k, r):
        ck = _splat(k)
        return (r[0] + (ck < v0).astype(jnp.int32),
                r[1] + (ck < v1).astype(jnp.int32))

    r = lax.fori_loop(0, base, count_le, (jnp.zeros((L,), jnp.int32),) * 2)
    for m in range(CHUNK):
        k = base + m
        ck = _splat(k)
        kv = jnp.full((L,), k, jnp.int32)
        inc0 = jnp.where(kv < j0, ck <= v0, ck < v0)
        inc1 = jnp.where(kv < j1, ck <= v1, ck < v1)
        r = (r[0] + inc0.astype(jnp.int32), r[1] + inc1.astype(jnp.int32))
    r = lax.fori_loop(base + CHUNK, CPAD, count_lt, r)

    rank_v[pl.ds(0, L)] = r[0]
    rank_v[pl.ds(L, L)] = r[1]
    pltpu.sync_copy(c_v.at[pl.ds(base, CHUNK)], out_hbm.at[rank_v])


def _make_main(bpw: int, unroll: int):
    @functools.partial(
        pl.kernel,
        out_type=jax.ShapeDtypeStruct((NW, L), jnp.float32),
        mesh=_mesh,
        compiler_params=_params,
        scratch_types=[
            pltpu.VMEM((CPAD,), jnp.float32),
            pltpu.VMEM((CPAD,), jnp.float32),
            pltpu.VMEM((bpw,), jnp.float32),
            pltpu.VMEM((bpw,), jnp.int32),
            pltpu.VMEM((L,), jnp.float32),
        ],
    )
    def _main(s_hbm, c_hbm, x_hbm, t_hbm, out_hbm, s_v, c_v, x_v, t_v, acc_v):
        wid = lax.axis_index("s") * NC + lax.axis_index("c")
        base = wid * bpw
        pltpu.sync_copy(s_hbm, s_v)
        pltpu.sync_copy(c_hbm, c_v)
        pltpu.sync_copy(x_hbm.at[pl.ds(base, bpw)], x_v)
        pltpu.sync_copy(t_hbm.at[pl.ds(base, bpw)], t_v)

        def one_vec(xv, tv):
            p = jnp.zeros((L,), jnp.int32)
            bit = CPAD // 2
            while bit:
                sv = plsc.load_gather(s_v, [p + (bit - 1)])
                p = jnp.where(sv <= xv, p + bit, p)
                bit //= 2
            i0 = p - 2
            i1 = p - 1
            d0 = jnp.abs(xv - plsc.load_gather(s_v, [jnp.maximum(i0, 0)]))
            d1 = jnp.abs(xv - plsc.load_gather(s_v, [jnp.maximum(i1, 0)]))
            d0 = jnp.where(i0 >= 0, d0, jnp.inf)
            d1 = jnp.where(i1 >= 0, d1, jnp.inf)
            d2 = jnp.abs(xv - plsc.load_gather(s_v, [p]))
            d3 = jnp.abs(xv - plsc.load_gather(s_v, [p + 1]))
            m1 = jnp.minimum(d1, d2)
            m2 = jnp.minimum(jnp.maximum(d1, d2), jnp.where(d1 <= d2, d0, d3))
            pull = jnp.abs(xv - plsc.load_gather(c_v, [tv]))
            push = jnp.where(pull > m1, m1, m2)
            return jnp.maximum(pull - push, 0.0)

        def body(i, acc):
            for u in range(unroll):
                off = (i * unroll + u) * L
                acc = acc + one_vec(x_v[pl.ds(off, L)], t_v[pl.ds(off, L)])
            return acc

        acc = lax.fori_loop(0, bpw // (L * unroll),
                            body, jnp.zeros((L,), jnp.float32))
        acc_v[...] = acc
        pltpu.sync_copy(acc_v, out_hbm.at[wid])

    return _main


def kernel(x, centers, transform_inds):
    b = x.shape[0]
    c = centers.shape[0]
    bpw = b // NW
    pad = jnp.full((CPAD - c,), jnp.inf, dtype=jnp.float32)
    c_pad = jnp.concatenate([centers.reshape(c), pad])
    s_sorted = _sort_centers(c_pad)
    partials = _make_main(bpw, 4)(
        s_sorted, c_pad, x.reshape(b), transform_inds)
    return jnp.sum(partials).reshape(1) / b
